# baseline (speedup 1.0000x reference)
import functools

import jax
import jax.numpy as jnp
from jax import lax
from jax.experimental import pallas as pl
from jax.experimental.pallas import tpu as pltpu
from jax.experimental.pallas import tpu_sc as plsc

NTOKEN = 1000000
NINP = 64
B = 16384
L = 50
N = B * L

_info = plsc.get_sparse_core_info()
NC = _info.num_cores
NS = _info.num_subcores
NW = NC * NS
BT = 128
NBT = B // BT
BLK_PER_W = NBT // NW
IDX_PER_BLK = BT * L


def _make_gather():
    mesh = plsc.VectorSubcoreMesh(core_axis_name="c", subcore_axis_name="s")

    @functools.partial(
        pl.kernel,
        mesh=mesh,
        out_type=jax.ShapeDtypeStruct((L, 8, BT, 8, BT), jnp.float32),
        scratch_types=[
            pltpu.VMEM((IDX_PER_BLK,), jnp.int32),
            pltpu.VMEM((L, BT), jnp.int32),
            pltpu.VMEM((BT, NINP), jnp.float32),
            pltpu.VMEM((BT, NINP), jnp.float32),
            pltpu.VMEM((8, 8, BT), jnp.float32),
            pltpu.VMEM((8, 8, BT), jnp.float32),
            pltpu.SemaphoreType.DMA,
            pltpu.SemaphoreType.DMA,
            pltpu.SemaphoreType.DMA,
            pltpu.SemaphoreType.DMA,
        ],
        compiler_params=pltpu.CompilerParams(use_tc_tiling_on_sc=False,
                                             needs_layout_passes=False),
    )
    def gather_kernel(idx_hbm, table_hbm, out_hbm, idxb, idxT,
                      rows0, rows1, t0, t1, sg0, sg1, sw0, sw1):
        wid = lax.axis_index("s") * NC + lax.axis_index("c")
        rows = (rows0, rows1)
        tbuf = (t0, t1)
        sg = (sg0, sg1)
        sw = (sw0, sw1)
        iota = lax.iota(jnp.int32, 16)
        rvec = tuple(iota + bg * 16 for bg in range(8))

        def fire_gather(l, par):
            pltpu.async_copy(table_hbm.at[idxT.at[l]], rows[par], sg[par])

        def wait_gather(par):
            pltpu.make_async_copy(table_hbm.at[idxT.at[0]], rows[par],
                                  sg[par]).wait()

        def transpose_group(par):
            r = rows[par]
            t = tbuf[par]

            @plsc.parallel_loop(0, NINP, step=1, unroll=16)
            def _(f):
                ft = f // 8
                fs = f % 8
                col = jnp.zeros((16,), jnp.int32) + f
                vs = [plsc.load_gather(r, [rvec[bg], col]) for bg in range(8)]
                for bg in range(8):
                    t[ft, fs, pl.ds(bg * 16, 16)] = vs[bg]

        def fire_write(l, bt, par):
            for ft in range(8):
                pltpu.async_copy(tbuf[par].at[ft], out_hbm.at[l, ft, bt],
                                 sw[par])

        def drain_write(l, bt, par):
            for ft in range(8):
                pltpu.make_async_copy(tbuf[par].at[ft],
                                      out_hbm.at[0, ft, 0], sw[par]).wait()

        def block(k, carry):
            bt = wid * BLK_PER_W + k
            pltpu.sync_copy(idx_hbm.at[pl.ds(bt * IDX_PER_BLK, IDX_PER_BLK)],
                            idxb)
            @plsc.parallel_loop(0, L, step=1, unroll=5)
            def _(l):
                for bg in range(8):
                    v = plsc.load_gather(idxb, [iota * L + (bg * 16 * L + l)])
                    idxT[l, pl.ds(bg * 16, 16)] = v

            fire_gather(0, 0)

            def body(i, carry2):
                for par in range(2):
                    l = i * 2 + par
                    nxt = l + 1

                    @pl.when(nxt < L)
                    def _():
                        fire_gather(nxt, 1 - par)

                    wait_gather(par)

                    @pl.when(k * L + l >= 2)
                    def _():
                        drain_write(0, 0, par)

                    transpose_group(par)
                    fire_write(l, bt, par)
                return carry2

            lax.fori_loop(0, L // 2, body, 0)
            return carry

        lax.fori_loop(0, BLK_PER_W, block, 0)

        for par in range(2):
            drain_write(0, 0, par)

    return gather_kernel


_gather = _make_gather()


def kernel(input, weight):
    idx_flat = input.reshape(N)
    v5 = _gather(idx_flat, weight)
    return v5.transpose(2, 4, 0, 1, 3).reshape(B, L, NINP)


# speedup vs baseline: 1.2036x; 1.2036x over previous
import functools

import jax
import jax.numpy as jnp
from jax import lax
from jax.experimental import pallas as pl
from jax.experimental.pallas import tpu as pltpu
from jax.experimental.pallas import tpu_sc as plsc

NTOKEN = 1000000
NINP = 64
B = 16384
L = 50
N = B * L

_info = plsc.get_sparse_core_info()
NC = _info.num_cores
NS = _info.num_subcores
NW = NC * NS
BT = 128
NBT = B // BT
BLK_PER_W = NBT // NW
IDX_PER_BLK = BT * L


def _make_gather():
    mesh = plsc.VectorSubcoreMesh(core_axis_name="c", subcore_axis_name="s")

    @functools.partial(
        pl.kernel,
        mesh=mesh,
        out_type=jax.ShapeDtypeStruct((L, 8, BT, 8, BT), jnp.float32),
        scratch_types=[
            pltpu.VMEM((IDX_PER_BLK,), jnp.int32),
            pltpu.VMEM((L, BT), jnp.int32),
            pltpu.VMEM((BT, NINP), jnp.float32),
            pltpu.VMEM((BT, NINP), jnp.float32),
            pltpu.VMEM((8, 8, BT), jnp.float32),
            pltpu.VMEM((8, 8, BT), jnp.float32),
            pltpu.SemaphoreType.DMA,
            pltpu.SemaphoreType.DMA,
            pltpu.SemaphoreType.DMA,
            pltpu.SemaphoreType.DMA,
        ],
        compiler_params=pltpu.CompilerParams(use_tc_tiling_on_sc=False,
                                             needs_layout_passes=False),
    )
    def gather_kernel(idx_hbm, table_hbm, out_hbm, idxb, idxT,
                      rows0, rows1, t0, t1, sg0, sg1, sw0, sw1):
        wid = lax.axis_index("s") * NC + lax.axis_index("c")
        rows = (rows0, rows1)
        tbuf = (t0, t1)
        sg = (sg0, sg1)
        sw = (sw0, sw1)
        iota = lax.iota(jnp.int32, 16)
        rvec = tuple(iota + bg * 16 for bg in range(8))

        def fire_gather(l, par):
            pltpu.async_copy(table_hbm.at[idxT.at[l]], rows[par], sg[par])

        def wait_gather(par):
            pltpu.make_async_copy(table_hbm.at[idxT.at[0]], rows[par],
                                  sg[par]).wait()

        def transpose_group(par):
            r = rows[par]
            t = tbuf[par]

            @plsc.parallel_loop(0, NINP, step=1, unroll=8)
            def _(f):
                ft = f // 8
                fs = f % 8
                col = jnp.zeros((16,), jnp.int32) + f
                for bg in range(8):
                    v = plsc.load_gather(r, [rvec[bg], col])
                    t[ft, fs, pl.ds(bg * 16, 16)] = v

        def fire_write(l, bt, par):
            for ft in range(8):
                pltpu.async_copy(tbuf[par].at[ft], out_hbm.at[l, ft, bt],
                                 sw[par])

        def drain_write(l, bt, par):
            for ft in range(8):
                pltpu.make_async_copy(tbuf[par].at[ft],
                                      out_hbm.at[0, ft, 0], sw[par]).wait()

        def block(k, carry):
            bt = wid * BLK_PER_W + k
            pltpu.sync_copy(idx_hbm.at[pl.ds(bt * IDX_PER_BLK, IDX_PER_BLK)],
                            idxb)
            @plsc.parallel_loop(0, L, step=1, unroll=5)
            def _(l):
                for bg in range(8):
                    v = plsc.load_gather(idxb, [iota * L + (bg * 16 * L + l)])
                    idxT[l, pl.ds(bg * 16, 16)] = v

            fire_gather(0, 0)

            def body(i, carry2):
                for par in range(2):
                    l = i * 2 + par
                    nxt = l + 1

                    @pl.when(nxt < L)
                    def _():
                        fire_gather(nxt, 1 - par)

                    wait_gather(par)

                    @pl.when(k * L + l >= 2)
                    def _():
                        drain_write(0, 0, par)

                    transpose_group(par)
                    fire_write(l, bt, par)
                return carry2

            lax.fori_loop(0, L // 2, body, 0)
            return carry

        lax.fori_loop(0, BLK_PER_W, block, 0)

        for par in range(2):
            drain_write(0, 0, par)

    return gather_kernel


_gather = _make_gather()


def kernel(input, weight):
    idx_flat = input.reshape(N)
    v5 = _gather(idx_flat, weight)
    return v5.transpose(2, 4, 0, 1, 3).reshape(B, L, NINP)


# speedup vs baseline: 1.9489x; 1.6193x over previous
import functools

import jax
import jax.numpy as jnp
from jax import lax
from jax.experimental import pallas as pl
from jax.experimental.pallas import tpu as pltpu
from jax.experimental.pallas import tpu_sc as plsc

NTOKEN = 1000000
NINP = 64
B = 16384
L = 50
N = B * L

_info = plsc.get_sparse_core_info()
NC = _info.num_cores
NS = _info.num_subcores
NW = NC * NS
BT = 128
NBT = B // BT
BLK_PER_W = NBT // NW
IDX_PER_BLK = BT * L


def _make_gather():
    mesh = plsc.VectorSubcoreMesh(core_axis_name="c", subcore_axis_name="s")

    @functools.partial(
        pl.kernel,
        mesh=mesh,
        out_type=jax.ShapeDtypeStruct((L, 8, BT, 8, BT), jnp.float32),
        scratch_types=[
            pltpu.VMEM((IDX_PER_BLK,), jnp.int32),
            pltpu.VMEM((L, BT), jnp.int32),
            pltpu.VMEM((BT, NINP), jnp.float32),
            pltpu.VMEM((BT, NINP), jnp.float32),
            pltpu.VMEM((NINP, BT + 1), jnp.float32),
            pltpu.VMEM((NINP, BT + 1), jnp.float32),
            pltpu.SemaphoreType.DMA,
            pltpu.SemaphoreType.DMA,
            pltpu.SemaphoreType.DMA,
            pltpu.SemaphoreType.DMA,
        ],
        compiler_params=pltpu.CompilerParams(use_tc_tiling_on_sc=False,
                                             needs_layout_passes=False),
    )
    def gather_kernel(idx_hbm, table_hbm, out_hbm, idxb, idxT,
                      rows0, rows1, t0, t1, sg0, sg1, sw0, sw1):
        wid = lax.axis_index("s") * NC + lax.axis_index("c")
        rows = (rows0, rows1)
        tbuf = (t0, t1)
        sg = (sg0, sg1)
        sw = (sw0, sw1)
        iota = lax.iota(jnp.int32, 16)
        fvec = tuple(iota + k * 16 for k in range(4))

        def fire_gather(l, par):
            pltpu.async_copy(table_hbm.at[idxT.at[l]], rows[par], sg[par])

        def wait_gather(par):
            pltpu.make_async_copy(table_hbm.at[idxT.at[0]], rows[par],
                                  sg[par]).wait()

        def transpose_group(par):
            r = rows[par]
            t = tbuf[par]

            @plsc.parallel_loop(0, BT, step=1, unroll=8)
            def _(b):
                col = jnp.zeros((16,), jnp.int32) + b
                for k in range(4):
                    v = r[b, pl.ds(k * 16, 16)]
                    plsc.store_scatter(t, [fvec[k], col], v)

        def fire_write(l, bt, par):
            for ft in range(8):
                pltpu.async_copy(
                    tbuf[par].at[pl.ds(ft * 8, 8), pl.ds(0, BT)],
                    out_hbm.at[l, ft, bt], sw[par])

        def drain_write(l, bt, par):
            for ft in range(8):
                pltpu.make_async_copy(
                    tbuf[par].at[pl.ds(ft * 8, 8), pl.ds(0, BT)],
                    out_hbm.at[0, ft, 0], sw[par]).wait()

        def block(k, carry):
            bt = wid * BLK_PER_W + k
            pltpu.sync_copy(idx_hbm.at[pl.ds(bt * IDX_PER_BLK, IDX_PER_BLK)],
                            idxb)
            @plsc.parallel_loop(0, L, step=1, unroll=5)
            def _(l):
                for bg in range(8):
                    v = plsc.load_gather(idxb, [iota * L + (bg * 16 * L + l)])
                    idxT[l, pl.ds(bg * 16, 16)] = v

            fire_gather(0, 0)

            def body(i, carry2):
                for par in range(2):
                    l = i * 2 + par
                    nxt = l + 1

                    @pl.when(nxt < L)
                    def _():
                        fire_gather(nxt, 1 - par)

                    wait_gather(par)

                    @pl.when(k * L + l >= 2)
                    def _():
                        drain_write(0, 0, par)

                    transpose_group(par)
                    fire_write(l, bt, par)
                return carry2

            lax.fori_loop(0, L // 2, body, 0)
            return carry

        lax.fori_loop(0, BLK_PER_W, block, 0)

        for par in range(2):
            drain_write(0, 0, par)

    return gather_kernel


_gather = _make_gather()


def kernel(input, weight):
    idx_flat = input.reshape(N)
    v5 = _gather(idx_flat, weight)
    return v5.transpose(2, 4, 0, 1, 3).reshape(B, L, NINP)


# speedup vs baseline: 2.1207x; 1.0882x over previous
import functools

import jax
import jax.numpy as jnp
from jax import lax
from jax.experimental import pallas as pl
from jax.experimental.pallas import tpu as pltpu
from jax.experimental.pallas import tpu_sc as plsc

NTOKEN = 1000000
NINP = 64
B = 16384
L = 50
N = B * L

_info = plsc.get_sparse_core_info()
NC = _info.num_cores
NS = _info.num_subcores
NW = NC * NS
BT = 128
NBT = B // BT
BLK_PER_W = NBT // NW
IDX_PER_BLK = BT * L


def _make_gather():
    mesh = plsc.VectorSubcoreMesh(core_axis_name="c", subcore_axis_name="s")

    @functools.partial(
        pl.kernel,
        mesh=mesh,
        out_type=jax.ShapeDtypeStruct((L, 8, BT, 8, BT), jnp.float32),
        scratch_types=[
            pltpu.VMEM((IDX_PER_BLK,), jnp.int32),
            pltpu.VMEM((L, BT), jnp.int32),
            pltpu.VMEM((BT, NINP), jnp.float32),
            pltpu.VMEM((BT, NINP), jnp.float32),
            pltpu.VMEM((NINP, BT + 1), jnp.float32),
            pltpu.VMEM((NINP, BT + 1), jnp.float32),
            pltpu.SemaphoreType.DMA,
            pltpu.SemaphoreType.DMA,
            pltpu.SemaphoreType.DMA,
            pltpu.SemaphoreType.DMA,
        ],
        compiler_params=pltpu.CompilerParams(use_tc_tiling_on_sc=False,
                                             needs_layout_passes=False),
    )
    def gather_kernel(idx_hbm, table_hbm, out_hbm, idxb, idxT,
                      rows0, rows1, t0, t1, sg0, sg1, sw0, sw1):
        wid = lax.axis_index("s") * NC + lax.axis_index("c")
        rows = (rows0, rows1)
        tbuf = (t0, t1)
        sg = (sg0, sg1)
        sw = (sw0, sw1)
        iota = lax.iota(jnp.int32, 16)
        fvec = tuple(iota + k * 16 for k in range(4))

        def fire_gather(l, par):
            pltpu.async_copy(table_hbm.at[idxT.at[l]], rows[par], sg[par])

        def wait_gather(par):
            pltpu.make_async_copy(table_hbm.at[idxT.at[0]], rows[par],
                                  sg[par]).wait()

        def transpose_group(par):
            r = rows[par]
            t = tbuf[par]

            @plsc.parallel_loop(0, BT, step=1, unroll=8)
            def _(b):
                col = jnp.zeros((16,), jnp.int32) + b
                for k in range(4):
                    v = r[b, pl.ds(k * 16, 16)]
                    plsc.store_scatter(t, [fvec[k], col], v)

        def fire_write(l, bt, par):
            for ft in range(8):
                pltpu.async_copy(
                    tbuf[par].at[pl.ds(ft * 8, 8), pl.ds(0, BT)],
                    out_hbm.at[l, ft, bt], sw[par])

        def drain_write(l, bt, par):
            for ft in range(8):
                pltpu.make_async_copy(
                    tbuf[par].at[pl.ds(ft * 8, 8), pl.ds(0, BT)],
                    out_hbm.at[0, ft, 0], sw[par]).wait()

        def block(k, carry):
            bt = wid * BLK_PER_W + k
            pltpu.sync_copy(idx_hbm.at[pl.ds(bt * IDX_PER_BLK, IDX_PER_BLK)],
                            idxb)
            @plsc.parallel_loop(0, L, step=1, unroll=5)
            def _(l):
                for bg in range(8):
                    v = plsc.load_gather(idxb, [iota * L + (bg * 16 * L + l)])
                    idxT[l, pl.ds(bg * 16, 16)] = v + v

            fire_gather(0, 0)

            def body(i, carry2):
                for par in range(2):
                    l = i * 2 + par
                    nxt = l + 1

                    @pl.when(nxt < L)
                    def _():
                        fire_gather(nxt, 1 - par)

                    wait_gather(par)

                    @pl.when(k * L + l >= 2)
                    def _():
                        drain_write(0, 0, par)

                    transpose_group(par)
                    fire_write(l, bt, par)
                return carry2

            lax.fori_loop(0, L // 2, body, 0)
            return carry

        lax.fori_loop(0, BLK_PER_W, block, 0)

        for par in range(2):
            drain_write(0, 0, par)

    return gather_kernel


_gather = _make_gather()


def kernel(input, weight):
    idx_flat = input.reshape(N)
    wpad = jnp.pad(weight, ((0, 0), (0, NINP))).reshape(2 * NTOKEN, NINP)
    v5 = _gather(idx_flat, wpad)
    return v5.transpose(2, 4, 0, 1, 3).reshape(B, L, NINP)
